# baseline (speedup 1.0000x reference)
import functools

import jax
import jax.numpy as jnp
from jax import lax
from jax.experimental import pallas as pl
from jax.experimental.pallas import tpu as pltpu
from jax.experimental.pallas import tpu_sc as plsc

NC = 2
NS = 16
LN = 16

N_NODES = 10000
N_EDGES = 320000
D = 128
DH = D // NC

NN = 10240
NPT = NN // NS
CB = 128

EPT = N_EDGES // NS
SCH = 400
SNCHUNK = EPT // SCH

STEPS = 3

_mesh = functools.partial(
    plsc.VectorSubcoreMesh,
    core_axis_name="c", subcore_axis_name="s",
    num_cores=NC, num_subcores=NS)


def _zero_fill_2d(ref, rows):
    zv = jnp.zeros((LN,), jnp.float32)

    def body(i, _):
        r = i // (DH // LN)
        c = (i % (DH // LN)) * LN
        ref[r, pl.ds(c, LN)] = zv
        return 0

    lax.fori_loop(0, rows * (DH // LN), body, 0)


def _fill_1d(ref, n, val):
    v = jnp.full((LN,), val, jnp.float32)

    def body(i, _):
        ref[pl.ds(i * LN, LN)] = v
        return 0

    lax.fori_loop(0, n // LN, body, 0)


def _smooth_body(xs, srcp, dstp, x0s, hA, hB,
                 acc, deg, sidx, didx, rows, ones, zb, tmp, inv, sem):
    cid = lax.axis_index("c")
    sid = lax.axis_index("s")
    nbase = sid * NPT

    _zero_fill_2d(zb, CB)
    _fill_1d(ones, SCH, 1.0)
    _fill_1d(inv, NPT, 0.0)
    pltpu.sync_copy(srcp.at[sid], sidx)
    pltpu.sync_copy(dstp.at[sid], didx)
    for k in range(NPT // CB):
        pltpu.sync_copy(zb, acc.at[pl.ds(nbase + k * CB, CB)])
        pltpu.sync_copy(inv.at[pl.ds(0, CB)], deg.at[pl.ds(nbase + k * CB, CB)])
    plsc.subcore_barrier()

    def edge_pass(table, with_deg):
        def cb(j, _):
            pltpu.async_copy(table.at[sidx.at[j]], rows, sem).wait()
            pltpu.sync_copy(rows, acc.at[didx.at[j]], add=True)
            if with_deg:
                pltpu.sync_copy(ones, deg.at[didx.at[j]], add=True)
            return 0
        lax.fori_loop(0, SNCHUNK, cb, 0)

    def _scale_rows_by_inv(k):
        def gbody(g, _):
            iv16 = inv[pl.ds(k * CB + g * LN, LN)]
            for e0 in range(LN):
                e = g * LN + e0
                iv = iv16[e0]
                for c in range(DH // LN):
                    rows[e, pl.ds(c * LN, LN)] = (
                        rows[e, pl.ds(c * LN, LN)] * iv)
            return 0
        lax.fori_loop(0, CB // LN, gbody, 0)

    def _add_tmp_into_rows():
        def abody(i, _):
            r = i // (DH // LN)
            c = (i % (DH // LN)) * LN
            rows[r, pl.ds(c, LN)] = rows[r, pl.ds(c, LN)] + tmp[r, pl.ds(c, LN)]
            return 0
        lax.fori_loop(0, CB * (DH // LN), abody, 0)

    def combine(h_out):
        for k in range(NPT // CB):
            base = nbase + k * CB
            pltpu.sync_copy(acc.at[pl.ds(base, CB)], rows.at[pl.ds(0, CB)])
            pltpu.sync_copy(zb, acc.at[pl.ds(base, CB)])
            _scale_rows_by_inv(k)
            pltpu.sync_copy(rows.at[pl.ds(0, CB)],
                            h_out.at[cid, pl.ds(base, CB)])

    def combine_final():
        for k in range(NPT // CB):
            base = nbase + k * CB
            pltpu.sync_copy(acc.at[pl.ds(base, CB)], rows.at[pl.ds(0, CB)])
            _scale_rows_by_inv(k)
            for src_tab in (xs, hA, hB):
                pltpu.sync_copy(src_tab.at[cid, pl.ds(base, CB)], tmp)
                _add_tmp_into_rows()

            def sbody(i, _):
                r = i // (DH // LN)
                c = (i % (DH // LN)) * LN
                rows[r, pl.ds(c, LN)] = (
                    rows[r, pl.ds(c, LN)] * (1.0 / (STEPS + 1)))
                return 0

            lax.fori_loop(0, CB * (DH // LN), sbody, 0)
            pltpu.sync_copy(rows.at[pl.ds(0, CB)],
                            x0s.at[cid, pl.ds(base, CB)])

    edge_pass(xs.at[cid], True)
    plsc.subcore_barrier()
    pltpu.sync_copy(deg.at[pl.ds(nbase, NPT)], inv)

    def ibody(i, _):
        v = inv[pl.ds(i * LN, LN)]
        inv[pl.ds(i * LN, LN)] = 1.0 / jnp.maximum(v, 1.0)
        return 0

    lax.fori_loop(0, NPT // LN, ibody, 0)
    combine(hA)
    plsc.subcore_barrier()

    edge_pass(hA.at[cid], False)
    plsc.subcore_barrier()
    combine(hB)
    plsc.subcore_barrier()

    edge_pass(hB.at[cid], False)
    plsc.subcore_barrier()
    combine_final()


def _smooth(xs, srcp, dstp):
    f32 = jnp.float32
    kern = pl.kernel(
        _smooth_body,
        out_type=[
            jax.ShapeDtypeStruct((NC, NN, DH), f32),
            jax.ShapeDtypeStruct((NC, NN, DH), f32),
            jax.ShapeDtypeStruct((NC, NN, DH), f32),
        ],
        mesh=_mesh(),
        compiler_params=pltpu.CompilerParams(use_tc_tiling_on_sc=False, needs_layout_passes=False),
        scratch_types=[
            pltpu.VMEM_SHARED((NN, DH), f32),
            pltpu.VMEM_SHARED((NN,), f32),
            pltpu.VMEM((SNCHUNK, SCH), jnp.int32),
            pltpu.VMEM((SNCHUNK, SCH), jnp.int32),
            pltpu.VMEM((SCH, DH), f32),
            pltpu.VMEM((SCH,), f32),
            pltpu.VMEM((CB, DH), f32),
            pltpu.VMEM((CB, DH), f32),
            pltpu.VMEM((NPT,), f32),
            pltpu.SemaphoreType.DMA,
        ],
    )
    x0s, _, _ = kern(xs, srcp, dstp)
    return x0s


def _conv_body(relu, ys, asrc, adst, srcp, dstp, zhbm, outs,
               acc, sidx, didx, rows, sat, dat, sem):
    cid = lax.axis_index("c")
    sid = lax.axis_index("s")
    nbase = sid * NPT

    pltpu.sync_copy(srcp.at[sid], sidx)
    pltpu.sync_copy(dstp.at[sid], didx)
    pltpu.sync_copy(asrc, sat)
    pltpu.sync_copy(adst, dat)
    for k in range(NPT // CB):
        pltpu.sync_copy(zhbm, acc.at[pl.ds(nbase + k * CB, CB)])
    plsc.subcore_barrier()

    def cb(j, _):
        pltpu.async_copy(ys.at[cid].at[sidx.at[j]], rows, sem).wait()

        def gbody(g, _):
            si = sidx[j, pl.ds(g * LN, LN)]
            di = didx[j, pl.ds(g * LN, LN)]
            t = plsc.load_gather(sat, [si]) + plsc.load_gather(dat, [di])
            sv = jnp.where(t >= 0.0, t, t * 0.01)
            for e0 in range(LN):
                e = g * LN + e0
                sval = sv[e0]
                for c in range(DH // LN):
                    rows[e, pl.ds(c * LN, LN)] = (
                        rows[e, pl.ds(c * LN, LN)] * sval)
            return 0

        lax.fori_loop(0, SCH // LN, gbody, 0)
        pltpu.sync_copy(rows, acc.at[didx.at[j]], add=True)
        return 0

    lax.fori_loop(0, SNCHUNK, cb, 0)
    plsc.subcore_barrier()

    if relu:
        for k in range(NPT // CB):
            base = nbase + k * CB
            pltpu.sync_copy(acc.at[pl.ds(base, CB)], rows.at[pl.ds(0, CB)])

            def rbody(i, _):
                r = i // (DH // LN)
                c = (i % (DH // LN)) * LN
                rows[r, pl.ds(c, LN)] = jnp.maximum(rows[r, pl.ds(c, LN)], 0.0)
                return 0

            lax.fori_loop(0, CB * (DH // LN), rbody, 0)
            pltpu.sync_copy(rows.at[pl.ds(0, CB)],
                            outs.at[cid, pl.ds(base, CB)])
    else:
        pltpu.sync_copy(acc.at[pl.ds(nbase, NPT)],
                        outs.at[cid, pl.ds(nbase, NPT)])


def _conv(ys, asrc, adst, srcp, dstp, zhbm, relu):
    f32 = jnp.float32
    kern = pl.kernel(
        functools.partial(_conv_body, relu),
        out_type=jax.ShapeDtypeStruct((NC, NN, DH), f32),
        mesh=_mesh(),
        compiler_params=pltpu.CompilerParams(use_tc_tiling_on_sc=False, needs_layout_passes=False),
        scratch_types=[
            pltpu.VMEM_SHARED((NN, DH), f32),
            pltpu.VMEM((SNCHUNK, SCH), jnp.int32),
            pltpu.VMEM((SNCHUNK, SCH), jnp.int32),
            pltpu.VMEM((SCH, DH), f32),
            pltpu.VMEM((NN,), f32),
            pltpu.VMEM((NN,), f32),
            pltpu.SemaphoreType.DMA,
        ],
    )
    return kern(ys, asrc, adst, srcp, dstp, zhbm)


def _dense_body(xs_ref, w_ref, wa_ref, y_ref, a_ref):
    X = jnp.concatenate([xs_ref[0], xs_ref[1]], axis=-1)
    Y = lax.dot_general(X, w_ref[...], (((1,), (1,)), ((), ())),
                        preferred_element_type=jnp.float32)
    y_ref[0] = Y[:, :DH]
    y_ref[1] = Y[:, DH:]
    a_ref[...] = lax.dot_general(X, wa_ref[...], (((1,), (0,)), ((), ())),
                                 preferred_element_type=jnp.float32)


def _dense(xs, W, waT):
    f32 = jnp.float32
    return pl.pallas_call(
        _dense_body,
        out_shape=[
            jax.ShapeDtypeStruct((NC, NN, DH), f32),
            jax.ShapeDtypeStruct((NN, 2), f32),
        ],
    )(xs, W, waT)


def kernel(x, edge_index, W1, Wa1, ba1, W2, Wa2, ba2):
    f32 = jnp.float32
    src = edge_index[0]
    dst = edge_index[1]
    srcp2 = src.reshape(NS, SNCHUNK, SCH)
    dstp2 = dst.reshape(NS, SNCHUNK, SCH)
    zhbm = jnp.zeros((CB, DH), f32)

    xs = jnp.zeros((NC, NN, DH), f32)
    xs = xs.at[0, :N_NODES].set(x[:, :DH]).at[1, :N_NODES].set(x[:, DH:])

    x0s = _smooth(xs, srcp2, dstp2)

    def attn_cols(Wa):
        return jnp.stack([Wa[0, :D], Wa[0, D:]], axis=1)

    y1s, A1 = _dense(x0s, W1, attn_cols(Wa1))
    a1s = A1[:, 0]
    a1d = A1[:, 1] + ba1[0]
    h1s = _conv(y1s, a1s, a1d, srcp2, dstp2, zhbm, relu=True)

    y2s, A2 = _dense(h1s, W2, attn_cols(Wa2))
    a2s = A2[:, 0]
    a2d = A2[:, 1] + ba2[0]
    outs = _conv(y2s, a2s, a2d, srcp2, dstp2, zhbm, relu=False)

    return jnp.concatenate([outs[0, :N_NODES], outs[1, :N_NODES]], axis=1)


# speedup vs baseline: 1.2606x; 1.2606x over previous
import functools

import jax
import jax.numpy as jnp
from jax import lax
from jax.experimental import pallas as pl
from jax.experimental.pallas import tpu as pltpu
from jax.experimental.pallas import tpu_sc as plsc

NC = 2
NS = 16
LN = 16

N_NODES = 10000
N_EDGES = 320000
D = 128
DH = D // NC

NN = 10240
NPT = NN // NS
CB = 128

EPT = N_EDGES // NS

CH = 160
NCHUNK = EPT // CH

SCH = 400
SNCHUNK = EPT // SCH

STEPS = 3

_mesh = functools.partial(
    plsc.VectorSubcoreMesh,
    core_axis_name="c", subcore_axis_name="s",
    num_cores=NC, num_subcores=NS)


def _zero_fill_2d(ref, rows):
    zv = jnp.zeros((LN,), jnp.float32)

    def body(i, _):
        r = i // (DH // LN)
        c = (i % (DH // LN)) * LN
        ref[r, pl.ds(c, LN)] = zv
        return 0

    lax.fori_loop(0, rows * (DH // LN), body, 0)


def _fill_1d(ref, n, val):
    v = jnp.full((LN,), val, jnp.float32)

    def body(i, _):
        ref[pl.ds(i * LN, LN)] = v
        return 0

    lax.fori_loop(0, n // LN, body, 0)


def _smooth_body(xs, srcp, dstp, x0s, hA, hB,
                 acc, deg, sidx, didx, rows, ones, zb, tmp, inv, sem):
    cid = lax.axis_index("c")
    sid = lax.axis_index("s")
    nbase = sid * NPT

    _zero_fill_2d(zb, CB)
    _fill_1d(ones, SCH, 1.0)
    _fill_1d(inv, NPT, 0.0)
    pltpu.sync_copy(srcp.at[sid], sidx)
    pltpu.sync_copy(dstp.at[sid], didx)
    for k in range(NPT // CB):
        pltpu.sync_copy(zb, acc.at[pl.ds(nbase + k * CB, CB)])
        pltpu.sync_copy(inv.at[pl.ds(0, CB)], deg.at[pl.ds(nbase + k * CB, CB)])
    plsc.subcore_barrier()

    def edge_pass(table, with_deg):
        def cb(j, _):
            pltpu.async_copy(table.at[sidx.at[j]], rows, sem).wait()
            pltpu.sync_copy(rows, acc.at[didx.at[j]], add=True)
            if with_deg:
                pltpu.sync_copy(ones, deg.at[didx.at[j]], add=True)
            return 0
        lax.fori_loop(0, SNCHUNK, cb, 0)

    def _scale_rows_by_inv(k):
        def gbody(g, _):
            iv16 = inv[pl.ds(k * CB + g * LN, LN)]
            for e0 in range(LN):
                e = g * LN + e0
                iv = iv16[e0]
                for c in range(DH // LN):
                    rows[e, pl.ds(c * LN, LN)] = (
                        rows[e, pl.ds(c * LN, LN)] * iv)
            return 0
        lax.fori_loop(0, CB // LN, gbody, 0)

    def _add_tmp_into_rows():
        def abody(i, _):
            r = i // (DH // LN)
            c = (i % (DH // LN)) * LN
            rows[r, pl.ds(c, LN)] = rows[r, pl.ds(c, LN)] + tmp[r, pl.ds(c, LN)]
            return 0
        lax.fori_loop(0, CB * (DH // LN), abody, 0)

    def combine(h_out):
        for k in range(NPT // CB):
            base = nbase + k * CB
            pltpu.sync_copy(acc.at[pl.ds(base, CB)], rows.at[pl.ds(0, CB)])
            pltpu.sync_copy(zb, acc.at[pl.ds(base, CB)])
            _scale_rows_by_inv(k)
            pltpu.sync_copy(rows.at[pl.ds(0, CB)],
                            h_out.at[cid, pl.ds(base, CB)])

    def combine_final():
        for k in range(NPT // CB):
            base = nbase + k * CB
            pltpu.sync_copy(acc.at[pl.ds(base, CB)], rows.at[pl.ds(0, CB)])
            _scale_rows_by_inv(k)
            for src_tab in (xs, hA, hB):
                pltpu.sync_copy(src_tab.at[cid, pl.ds(base, CB)], tmp)
                _add_tmp_into_rows()

            def sbody(i, _):
                r = i // (DH // LN)
                c = (i % (DH // LN)) * LN
                rows[r, pl.ds(c, LN)] = (
                    rows[r, pl.ds(c, LN)] * (1.0 / (STEPS + 1)))
                return 0

            lax.fori_loop(0, CB * (DH // LN), sbody, 0)
            pltpu.sync_copy(rows.at[pl.ds(0, CB)],
                            x0s.at[cid, pl.ds(base, CB)])

    edge_pass(xs.at[cid], True)
    plsc.subcore_barrier()
    pltpu.sync_copy(deg.at[pl.ds(nbase, NPT)], inv)

    def ibody(i, _):
        v = inv[pl.ds(i * LN, LN)]
        inv[pl.ds(i * LN, LN)] = 1.0 / jnp.maximum(v, 1.0)
        return 0

    lax.fori_loop(0, NPT // LN, ibody, 0)
    combine(hA)
    plsc.subcore_barrier()

    edge_pass(hA.at[cid], False)
    plsc.subcore_barrier()
    combine(hB)
    plsc.subcore_barrier()

    edge_pass(hB.at[cid], False)
    plsc.subcore_barrier()
    combine_final()


def _smooth(xs, srcp, dstp):
    f32 = jnp.float32
    kern = pl.kernel(
        _smooth_body,
        out_type=[
            jax.ShapeDtypeStruct((NC, NN, DH), f32),
            jax.ShapeDtypeStruct((NC, NN, DH), f32),
            jax.ShapeDtypeStruct((NC, NN, DH), f32),
        ],
        mesh=_mesh(),
        compiler_params=pltpu.CompilerParams(use_tc_tiling_on_sc=False, needs_layout_passes=False),
        scratch_types=[
            pltpu.VMEM_SHARED((NN, DH), f32),
            pltpu.VMEM_SHARED((NN,), f32),
            pltpu.VMEM((SNCHUNK, SCH), jnp.int32),
            pltpu.VMEM((SNCHUNK, SCH), jnp.int32),
            pltpu.VMEM((SCH, DH), f32),
            pltpu.VMEM((SCH,), f32),
            pltpu.VMEM((CB, DH), f32),
            pltpu.VMEM((CB, DH), f32),
            pltpu.VMEM((NPT,), f32),
            pltpu.SemaphoreType.DMA,
        ],
    )
    x0s, _, _ = kern(xs, srcp, dstp)
    return x0s


def _conv_body(relu, ys, asrc, adst, srcp, dstp, outs,
               acc, sidx, didx, rows, zb, sat, dat, sem):
    cid = lax.axis_index("c")
    sid = lax.axis_index("s")
    nbase = sid * NPT

    _zero_fill_2d(zb, CB)
    pltpu.sync_copy(srcp.at[sid], sidx)
    pltpu.sync_copy(dstp.at[sid], didx)
    pltpu.sync_copy(asrc, sat)
    pltpu.sync_copy(adst, dat)
    for k in range(NPT // CB):
        pltpu.sync_copy(zb, acc.at[pl.ds(nbase + k * CB, CB)])
    plsc.subcore_barrier()

    def cb(j, _):
        pltpu.async_copy(ys.at[cid].at[sidx.at[j]], rows, sem).wait()

        def gbody(g, _):
            si = sidx[j, pl.ds(g * LN, LN)]
            di = didx[j, pl.ds(g * LN, LN)]
            t = plsc.load_gather(sat, [si]) + plsc.load_gather(dat, [di])
            sv = jnp.where(t >= 0.0, t, t * 0.01)
            for e0 in range(LN):
                e = g * LN + e0
                sval = sv[e0]
                for c in range(DH // LN):
                    rows[e, pl.ds(c * LN, LN)] = (
                        rows[e, pl.ds(c * LN, LN)] * sval)
            return 0

        lax.fori_loop(0, CH // LN, gbody, 0)
        pltpu.sync_copy(rows, acc.at[didx.at[j]], add=True)
        return 0

    lax.fori_loop(0, NCHUNK, cb, 0)
    plsc.subcore_barrier()

    if relu:
        for k in range(NPT // CB):
            base = nbase + k * CB
            pltpu.sync_copy(acc.at[pl.ds(base, CB)], rows.at[pl.ds(0, CB)])

            def rbody(i, _):
                r = i // (DH // LN)
                c = (i % (DH // LN)) * LN
                rows[r, pl.ds(c, LN)] = jnp.maximum(rows[r, pl.ds(c, LN)], 0.0)
                return 0

            lax.fori_loop(0, CB * (DH // LN), rbody, 0)
            pltpu.sync_copy(rows.at[pl.ds(0, CB)],
                            outs.at[cid, pl.ds(base, CB)])
    else:
        pltpu.sync_copy(acc.at[pl.ds(nbase, NPT)],
                        outs.at[cid, pl.ds(nbase, NPT)])


def _conv(ys, asrc, adst, srcp, dstp, relu):
    f32 = jnp.float32
    kern = pl.kernel(
        functools.partial(_conv_body, relu),
        out_type=jax.ShapeDtypeStruct((NC, NN, DH), f32),
        mesh=_mesh(),
        compiler_params=pltpu.CompilerParams(use_tc_tiling_on_sc=False, needs_layout_passes=False),
        scratch_types=[
            pltpu.VMEM_SHARED((NN, DH), f32),
            pltpu.VMEM((NCHUNK, CH), jnp.int32),
            pltpu.VMEM((NCHUNK, CH), jnp.int32),
            pltpu.VMEM((CH, DH), f32),
            pltpu.VMEM((CB, DH), f32),
            pltpu.VMEM((NN,), f32),
            pltpu.VMEM((NN,), f32),
            pltpu.SemaphoreType.DMA,
        ],
    )
    return kern(ys, asrc, adst, srcp, dstp)


def _dense_body(xs_ref, w_ref, wa_ref, y_ref, a_ref):
    X = jnp.concatenate([xs_ref[0], xs_ref[1]], axis=-1)
    Y = lax.dot_general(X, w_ref[...], (((1,), (1,)), ((), ())),
                        preferred_element_type=jnp.float32)
    y_ref[0] = Y[:, :DH]
    y_ref[1] = Y[:, DH:]
    a_ref[...] = lax.dot_general(X, wa_ref[...], (((1,), (0,)), ((), ())),
                                 preferred_element_type=jnp.float32)


def _dense(xs, W, waT):
    f32 = jnp.float32
    return pl.pallas_call(
        _dense_body,
        out_shape=[
            jax.ShapeDtypeStruct((NC, NN, DH), f32),
            jax.ShapeDtypeStruct((NN, 2), f32),
        ],
    )(xs, W, waT)


def kernel(x, edge_index, W1, Wa1, ba1, W2, Wa2, ba2):
    f32 = jnp.float32
    src = edge_index[0]
    dst = edge_index[1]
    srcp = src.reshape(NS, NCHUNK, CH)
    dstp = dst.reshape(NS, NCHUNK, CH)
    srcp2 = src.reshape(NS, SNCHUNK, SCH)
    dstp2 = dst.reshape(NS, SNCHUNK, SCH)

    xs = jnp.zeros((NC, NN, DH), f32)
    xs = xs.at[0, :N_NODES].set(x[:, :DH]).at[1, :N_NODES].set(x[:, DH:])

    x0s = _smooth(xs, srcp2, dstp2)

    def attn_cols(Wa):
        return jnp.stack([Wa[0, :D], Wa[0, D:]], axis=1)

    y1s, A1 = _dense(x0s, W1, attn_cols(Wa1))
    a1s = A1[:, 0]
    a1d = A1[:, 1] + ba1[0]
    h1s = _conv(y1s, a1s, a1d, srcp, dstp, relu=True)

    y2s, A2 = _dense(h1s, W2, attn_cols(Wa2))
    a2s = A2[:, 0]
    a2d = A2[:, 1] + ba2[0]
    outs = _conv(y2s, a2s, a2d, srcp, dstp, relu=False)

    return jnp.concatenate([outs[0, :N_NODES], outs[1, :N_NODES]], axis=1)
